# initial kernel scaffold (unmeasured)
import jax
import jax.numpy as jnp
from jax import lax
from jax.experimental import pallas as pl
from jax.experimental.pallas import tpu as pltpu

N_DEV = 4


def kernel(x, w_mat):
    m, k_shard = x.shape
    _, n = w_mat.shape
    mc = m // N_DEV

    def body(x_ref, w_ref, out_ref, recv_ref,
             rs_send_sems, rs_recv_sems, ag_send_sems, ag_recv_sems):
        me = lax.axis_index("i")
        left = (me + N_DEV - 1) % N_DEV
        right = (me + 1) % N_DEV

        barrier_sem = pltpu.get_barrier_semaphore()
        for nbr in (left, right):
            pl.semaphore_signal(barrier_sem, inc=1, device_id=(nbr,),
                                device_id_type=pl.DeviceIdType.MESH)
        pl.semaphore_wait(barrier_sem, 2)

        out_ref[:, :] = jnp.dot(x_ref[:, :], w_ref[:, :],
                                preferred_element_type=jnp.float32)

        for s in range(N_DEV - 1):
            c_send = (me - s) % N_DEV
            c_recv = (me - s - 1) % N_DEV
            rdma = pltpu.make_async_remote_copy(
                src_ref=out_ref.at[pl.ds(c_send * mc, mc), :],
                dst_ref=recv_ref.at[s],
                send_sem=rs_send_sems.at[s],
                recv_sem=rs_recv_sems.at[s],
                device_id=(right,),
                device_id_type=pl.DeviceIdType.MESH,
            )
            rdma.start()
            rdma.wait()
            out_ref[pl.ds(c_recv * mc, mc), :] = (
                out_ref[pl.ds(c_recv * mc, mc), :] + recv_ref[s, :, :]
            )

        for s in range(N_DEV - 1):
            g = (me + 1 - s) % N_DEV
            rdma = pltpu.make_async_remote_copy(
                src_ref=out_ref.at[pl.ds(g * mc, mc), :],
                dst_ref=out_ref.at[pl.ds(g * mc, mc), :],
                send_sem=ag_send_sems.at[s],
                recv_sem=ag_recv_sems.at[s],
                device_id=(right,),
                device_id_type=pl.DeviceIdType.MESH,
            )
            rdma.start()
            rdma.wait()

        y = jnp.maximum(out_ref[:, :], 0.0)
        scale = jnp.max(jnp.abs(y)) / 127.0
        q = jnp.clip(jnp.round(y / scale), -127.0, 127.0)
        out_ref[:, :] = q * scale

    return pl.pallas_call(
        body,
        out_shape=jax.ShapeDtypeStruct((m, n), jnp.float32),
        in_specs=[
            pl.BlockSpec(memory_space=pltpu.VMEM),
            pl.BlockSpec(memory_space=pltpu.VMEM),
        ],
        out_specs=pl.BlockSpec(memory_space=pltpu.VMEM),
        scratch_shapes=[
            pltpu.VMEM((N_DEV - 1, mc, n), jnp.float32),
            pltpu.SemaphoreType.DMA((N_DEV - 1,)),
            pltpu.SemaphoreType.DMA((N_DEV - 1,)),
            pltpu.SemaphoreType.DMA((N_DEV - 1,)),
            pltpu.SemaphoreType.DMA((N_DEV - 1,)),
        ],
        compiler_params=pltpu.CompilerParams(collective_id=0),
    )(x, w_mat)


# baseline (device time: 636190 ns/iter reference)
import jax
import jax.numpy as jnp
from jax import lax
from jax.experimental import pallas as pl
from jax.experimental.pallas import tpu as pltpu

N_DEV = 4


def kernel(x, w_mat):
    m, k_shard = x.shape
    _, n = w_mat.shape
    mc = m // N_DEV

    def body(x_ref, w_ref, out_ref, x_stage, rs_recv, copy_sem,
             rs_send_sems, rs_recv_sems, ag_send_sems, ag_recv_sems,
             credit_sem):
        me = lax.axis_index("i")
        left = (me + N_DEV - 1) % N_DEV
        right = (me + 1) % N_DEV

        barrier_sem = pltpu.get_barrier_semaphore()
        for nbr in (left, right):
            pl.semaphore_signal(barrier_sem, inc=1, device_id=(nbr,),
                                device_id_type=pl.DeviceIdType.MESH)
        pl.semaphore_wait(barrier_sem, 2)

        for r in range(N_DEV):
            cp = pltpu.make_async_copy(
                x_ref.at[pl.ds(r * mc, mc), :], x_stage, copy_sem)
            cp.start()
            cp.wait()
            out_ref[pl.ds(r * mc, mc), :] = jnp.dot(
                x_stage[:, :], w_ref[:, :],
                preferred_element_type=jnp.float32)

        for s in range(N_DEV - 1):
            c_send = (me - s) % N_DEV
            c_recv = (me - s - 1) % N_DEV
            if s > 0:
                pl.semaphore_wait(credit_sem, 1)
            rdma = pltpu.make_async_remote_copy(
                src_ref=out_ref.at[pl.ds(c_send * mc, mc), :],
                dst_ref=rs_recv,
                send_sem=rs_send_sems.at[s],
                recv_sem=rs_recv_sems.at[s],
                device_id=(right,),
                device_id_type=pl.DeviceIdType.MESH,
            )
            rdma.start()
            rdma.wait()
            out_ref[pl.ds(c_recv * mc, mc), :] = (
                out_ref[pl.ds(c_recv * mc, mc), :] + rs_recv[:, :]
            )
            if s < N_DEV - 2:
                pl.semaphore_signal(credit_sem, inc=1, device_id=(left,),
                                    device_id_type=pl.DeviceIdType.MESH)

        for s in range(N_DEV - 1):
            g = (me + 1 - s) % N_DEV
            rdma = pltpu.make_async_remote_copy(
                src_ref=out_ref.at[pl.ds(g * mc, mc), :],
                dst_ref=out_ref.at[pl.ds(g * mc, mc), :],
                send_sem=ag_send_sems.at[s],
                recv_sem=ag_recv_sems.at[s],
                device_id=(right,),
                device_id_type=pl.DeviceIdType.MESH,
            )
            rdma.start()
            rdma.wait()

        amax = jnp.float32(0.0)
        for r in range(N_DEV):
            amax = jnp.maximum(amax, jnp.max(out_ref[pl.ds(r * mc, mc), :]))
        scale = amax / 127.0
        for r in range(N_DEV):
            y = jnp.maximum(out_ref[pl.ds(r * mc, mc), :], 0.0)
            q = jnp.clip(jnp.round(y / scale), -127.0, 127.0)
            out_ref[pl.ds(r * mc, mc), :] = q * scale

    return pl.pallas_call(
        body,
        out_shape=jax.ShapeDtypeStruct((m, n), jnp.float32),
        in_specs=[
            pl.BlockSpec(memory_space=pl.ANY),
            pl.BlockSpec(memory_space=pltpu.VMEM),
        ],
        out_specs=pl.BlockSpec(memory_space=pltpu.VMEM),
        scratch_shapes=[
            pltpu.VMEM((mc, k_shard), jnp.float32),
            pltpu.VMEM((mc, n), jnp.float32),
            pltpu.SemaphoreType.DMA,
            pltpu.SemaphoreType.DMA((N_DEV - 1,)),
            pltpu.SemaphoreType.DMA((N_DEV - 1,)),
            pltpu.SemaphoreType.DMA((N_DEV - 1,)),
            pltpu.SemaphoreType.DMA((N_DEV - 1,)),
            pltpu.SemaphoreType.REGULAR,
        ],
        compiler_params=pltpu.CompilerParams(
            collective_id=0,
            vmem_limit_bytes=63 * 1024 * 1024,
        ),
    )(x, w_mat)


# device time: 241919 ns/iter; 2.6298x vs baseline; 2.6298x over previous
import jax
import jax.numpy as jnp
from jax import lax
from jax.experimental import pallas as pl
from jax.experimental.pallas import tpu as pltpu

N_DEV = 4


def kernel(x, w_mat):
    m, k_shard = x.shape
    _, n = w_mat.shape
    mc = m // N_DEV
    nh = n // 2

    def body(x_ref, w_ref, out_ref, x_stage, recv_a, recv_b, q_ref, amax_ref,
             copy_sem, rsa_send, rsa_recv, rsb_send, rsb_recv,
             aga_send, aga_recv, agb_send, agb_recv, amx_send, amx_recv,
             credit_a, credit_b):
        me = lax.axis_index("i")
        left = (me + N_DEV - 1) % N_DEV
        right = (me + 1) % N_DEV

        barrier_sem = pltpu.get_barrier_semaphore()
        for nbr in (left, right):
            pl.semaphore_signal(barrier_sem, inc=1, device_id=(nbr,),
                                device_id_type=pl.DeviceIdType.MESH)
        pl.semaphore_wait(barrier_sem, 2)

        def gemm_chunk(c):
            for h in range(2):
                r0 = c * mc + h * (mc // 2)
                cp = pltpu.make_async_copy(
                    x_ref.at[pl.ds(r0, mc // 2), :], x_stage, copy_sem)
                cp.start()
                cp.wait()
                out_ref[pl.ds(r0, mc // 2), :] = jnp.dot(
                    x_stage[:, :], w_ref[:, :],
                    preferred_element_type=jnp.float32)

        gemm_chunk(me)

        for s in range(N_DEV - 1):
            ca_send = (me - s) % N_DEV
            ca_recv = (me - s - 1) % N_DEV
            cb_send = (me + s) % N_DEV
            cb_recv = (me + s + 1) % N_DEV
            if s > 0:
                pl.semaphore_wait(credit_a, 1)
                pl.semaphore_wait(credit_b, 1)
            rdma_a = pltpu.make_async_remote_copy(
                src_ref=out_ref.at[pl.ds(ca_send * mc, mc), pl.ds(0, nh)],
                dst_ref=recv_a,
                send_sem=rsa_send.at[s], recv_sem=rsa_recv.at[s],
                device_id=(right,), device_id_type=pl.DeviceIdType.MESH)
            rdma_b = pltpu.make_async_remote_copy(
                src_ref=out_ref.at[pl.ds(cb_send * mc, mc), pl.ds(nh, nh)],
                dst_ref=recv_b,
                send_sem=rsb_send.at[s], recv_sem=rsb_recv.at[s],
                device_id=(left,), device_id_type=pl.DeviceIdType.MESH)
            rdma_a.start()
            rdma_b.start()
            if s == 0:
                gemm_chunk((me + N_DEV - 1) % N_DEV)
                gemm_chunk((me + 1) % N_DEV)
            elif s == 1:
                gemm_chunk((me + 2) % N_DEV)
            rdma_a.wait()
            out_ref[pl.ds(ca_recv * mc, mc), pl.ds(0, nh)] = (
                out_ref[pl.ds(ca_recv * mc, mc), pl.ds(0, nh)] + recv_a[:, :])
            rdma_b.wait()
            out_ref[pl.ds(cb_recv * mc, mc), pl.ds(nh, nh)] = (
                out_ref[pl.ds(cb_recv * mc, mc), pl.ds(nh, nh)] + recv_b[:, :])
            if s < N_DEV - 2:
                pl.semaphore_signal(credit_a, inc=1, device_id=(left,),
                                    device_id_type=pl.DeviceIdType.MESH)
                pl.semaphore_signal(credit_b, inc=1, device_id=(right,),
                                    device_id_type=pl.DeviceIdType.MESH)

        own_a = (me + 1) % N_DEV
        own_b = (me + N_DEV - 1) % N_DEV

        amax_loc = jnp.maximum(
            jnp.max(out_ref[pl.ds(own_a * mc, mc), pl.ds(0, nh)]),
            jnp.max(out_ref[pl.ds(own_b * mc, mc), pl.ds(nh, nh)]))
        amax_loc = jnp.maximum(amax_loc, 0.0)
        amax_ref[pl.ds(me, 1), :, :] = jnp.full(
            (1, 8, 128), amax_loc, jnp.float32)
        for s in range(N_DEV - 1):
            slot = (me - s) % N_DEV
            rdma = pltpu.make_async_remote_copy(
                src_ref=amax_ref.at[slot], dst_ref=amax_ref.at[slot],
                send_sem=amx_send.at[s], recv_sem=amx_recv.at[s],
                device_id=(right,), device_id_type=pl.DeviceIdType.MESH)
            rdma.start()
            rdma.wait()
        scale = jnp.max(amax_ref[:, :, :]) / 127.0

        def quant(c, col0):
            y = jnp.maximum(out_ref[pl.ds(c * mc, mc), pl.ds(col0, nh)], 0.0)
            q = jnp.clip(jnp.round(y / scale), -127.0, 127.0)
            q_ref[pl.ds(c * mc, mc), pl.ds(col0, nh)] = q.astype(jnp.int8)

        def dequant(c, col0):
            out_ref[pl.ds(c * mc, mc), pl.ds(col0, nh)] = (
                q_ref[pl.ds(c * mc, mc), pl.ds(col0, nh)].astype(jnp.float32)
                * scale)

        quant(own_a, 0)
        quant(own_b, nh)

        for s in range(N_DEV - 1):
            ga = (me + 1 - s) % N_DEV
            gb = (me + N_DEV - 1 + s) % N_DEV
            rdma_a = pltpu.make_async_remote_copy(
                src_ref=q_ref.at[pl.ds(ga * mc, mc), pl.ds(0, nh)],
                dst_ref=q_ref.at[pl.ds(ga * mc, mc), pl.ds(0, nh)],
                send_sem=aga_send.at[s], recv_sem=aga_recv.at[s],
                device_id=(right,), device_id_type=pl.DeviceIdType.MESH)
            rdma_b = pltpu.make_async_remote_copy(
                src_ref=q_ref.at[pl.ds(gb * mc, mc), pl.ds(nh, nh)],
                dst_ref=q_ref.at[pl.ds(gb * mc, mc), pl.ds(nh, nh)],
                send_sem=agb_send.at[s], recv_sem=agb_recv.at[s],
                device_id=(left,), device_id_type=pl.DeviceIdType.MESH)
            rdma_a.start()
            rdma_b.start()
            if s == 0:
                dequant(own_a, 0)
                dequant(own_b, nh)
            else:
                dequant((me - s + 1) % N_DEV, 0)
                dequant((me + s - 1) % N_DEV, nh)
            rdma_a.wait()
            rdma_b.wait()
        dequant((me + 2) % N_DEV, 0)
        dequant((me + 2) % N_DEV, nh)

    return pl.pallas_call(
        body,
        out_shape=jax.ShapeDtypeStruct((m, n), jnp.float32),
        in_specs=[
            pl.BlockSpec(memory_space=pl.ANY),
            pl.BlockSpec(memory_space=pltpu.VMEM),
        ],
        out_specs=pl.BlockSpec(memory_space=pltpu.VMEM),
        scratch_shapes=[
            pltpu.VMEM((mc // 2, k_shard), jnp.float32),
            pltpu.VMEM((mc, nh), jnp.float32),
            pltpu.VMEM((mc, nh), jnp.float32),
            pltpu.VMEM((m, n), jnp.int8),
            pltpu.VMEM((N_DEV, 8, 128), jnp.float32),
            pltpu.SemaphoreType.DMA,
            pltpu.SemaphoreType.DMA((N_DEV - 1,)),
            pltpu.SemaphoreType.DMA((N_DEV - 1,)),
            pltpu.SemaphoreType.DMA((N_DEV - 1,)),
            pltpu.SemaphoreType.DMA((N_DEV - 1,)),
            pltpu.SemaphoreType.DMA((N_DEV - 1,)),
            pltpu.SemaphoreType.DMA((N_DEV - 1,)),
            pltpu.SemaphoreType.DMA((N_DEV - 1,)),
            pltpu.SemaphoreType.DMA((N_DEV - 1,)),
            pltpu.SemaphoreType.DMA((N_DEV - 1,)),
            pltpu.SemaphoreType.DMA((N_DEV - 1,)),
            pltpu.SemaphoreType.REGULAR,
            pltpu.SemaphoreType.REGULAR,
        ],
        compiler_params=pltpu.CompilerParams(
            collective_id=0,
            vmem_limit_bytes=63 * 1024 * 1024,
        ),
    )(x, w_mat)
